# Initial kernel scaffold; baseline (speedup 1.0000x reference)
import jax
import jax.numpy as jnp
from jax.experimental import pallas as pl


def kernel(X):
    raise NotImplementedError("write your pallas kernel here")


# speedup vs baseline: 2.1638x; 2.1638x over previous
import jax
import jax.numpy as jnp
from jax.experimental import pallas as pl
from jax.experimental.pallas import tpu as pltpu

N_ROWS = 16384
N_COLS = 2048
BLK = 256
NBLK = N_ROWS // BLK


def _fused(x_ref, o_ref, acc_sum, acc_sq, g_scr):
    i = pl.program_id(0)

    @pl.when(i == 0)
    def _init():
        acc_sum[...] = jnp.zeros_like(acc_sum)
        acc_sq[...] = jnp.zeros_like(acc_sq)

    @pl.when(i < NBLK)
    def _accum():
        x = x_ref[...]
        xr = x.reshape(BLK // 8, 8, N_COLS)
        acc_sum[...] += jnp.sum(xr, axis=0)
        acc_sq[...] += jnp.sum(xr * xr, axis=0)

    @pl.when(i == NBLK - 1)
    def _select():
        s = jnp.sum(acc_sum[...], axis=0, keepdims=True)
        q = jnp.sum(acc_sq[...], axis=0, keepdims=True)
        n = float(N_ROWS)
        var = (q - s * s / n) / (n - 1.0)
        m = (var > 0.0).astype(jnp.float32)
        cs = m
        sh = 1
        while sh < N_COLS:
            cs = cs + jnp.concatenate(
                [jnp.zeros((1, sh), jnp.float32), cs[:, :-sh]], axis=1
            )
            sh *= 2
        pos = (cs - 1.0).astype(jnp.int32)
        cnt = cs[:, N_COLS - 1:].astype(jnp.int32)
        io0 = jax.lax.broadcasted_iota(jnp.int32, (N_COLS, N_COLS), 0)
        io1 = jax.lax.broadcasted_iota(jnp.int32, (N_COLS, N_COLS), 1)
        keep = (io0 == pos) & (m > 0.0)
        pad = (io1 == 0) & (io0 >= cnt)
        gt = jnp.where(keep | pad, 1.0, 0.0)
        g_scr[...] = gt.T

    @pl.when(i >= NBLK)
    def _emit():
        o_ref[...] = jnp.dot(
            x_ref[...], g_scr[...], preferred_element_type=jnp.float32
        )


def kernel(X):
    return pl.pallas_call(
        _fused,
        grid=(2 * NBLK,),
        in_specs=[
            pl.BlockSpec(
                (BLK, N_COLS), lambda i: (jnp.where(i < NBLK, i, i - NBLK), 0)
            )
        ],
        out_specs=pl.BlockSpec((BLK, N_COLS), lambda i: (jnp.maximum(i - NBLK, 0), 0)),
        out_shape=jax.ShapeDtypeStruct((N_ROWS, N_COLS), jnp.float32),
        scratch_shapes=[
            pltpu.VMEM((8, N_COLS), jnp.float32),
            pltpu.VMEM((8, N_COLS), jnp.float32),
            pltpu.VMEM((N_COLS, N_COLS), jnp.float32),
        ],
        compiler_params=pltpu.CompilerParams(
            dimension_semantics=("arbitrary",),
        ),
    )(X)


# speedup vs baseline: 5.0310x; 2.3251x over previous
import jax
import jax.numpy as jnp
from jax.experimental import pallas as pl
from jax.experimental.pallas import tpu as pltpu

N_ROWS = 16384
N_COLS = 2048
BLK = 256
NBLK = N_ROWS // BLK
NF = float(N_ROWS)


def _pass1(x_ref, o_ref, var_ref, flag_ref, acc_sum, acc_sq):
    i = pl.program_id(0)

    @pl.when(i == 0)
    def _init():
        acc_sum[...] = jnp.zeros_like(acc_sum)
        acc_sq[...] = jnp.zeros_like(acc_sq)

    x = x_ref[...]
    o_ref[...] = x
    xr = x.reshape(BLK // 8, 8, N_COLS)
    acc_sum[...] += jnp.sum(xr, axis=0)
    acc_sq[...] += jnp.sum(xr * xr, axis=0)

    @pl.when(i == NBLK - 1)
    def _finish():
        s = jnp.sum(acc_sum[...], axis=0, keepdims=True)
        q = jnp.sum(acc_sq[...], axis=0, keepdims=True)
        var = (q - s * s / NF) / (NF - 1.0)
        var_ref[...] = var
        m = (var > 0.0).astype(jnp.float32)
        cnt = jnp.sum(m, axis=1, keepdims=True)
        flag_ref[...] = (cnt == float(N_COLS)).astype(jnp.int32)


def _pass2(x_hbm, spec_hbm, var_ref, flag_ref, o_hbm, xb, ob, gb, semx, semo):
    i = pl.program_id(0)
    slow = flag_ref[0, 0] == 0

    @pl.when(slow & (i == 0))
    def _build_g():
        var = var_ref[...]
        m = (var > 0.0).astype(jnp.float32)
        cs = m
        sh = 1
        while sh < N_COLS:
            cs = cs + jnp.concatenate(
                [jnp.zeros((1, sh), jnp.float32), cs[:, :-sh]], axis=1
            )
            sh *= 2
        pos = (cs - 1.0).astype(jnp.int32)
        cnt = cs[:, N_COLS - 1:].astype(jnp.int32)
        io0 = jax.lax.broadcasted_iota(jnp.int32, (N_COLS, N_COLS), 0)
        io1 = jax.lax.broadcasted_iota(jnp.int32, (N_COLS, N_COLS), 1)
        keep = (io0 == pos) & (m > 0.0)
        pad = (io1 == 0) & (io0 >= cnt)
        gt = jnp.where(keep | pad, 1.0, 0.0)
        gb[...] = gt.T

    @pl.when(slow)
    def _gather_block():
        cp_in = pltpu.make_async_copy(
            x_hbm.at[pl.ds(i * BLK, BLK), :], xb, semx
        )
        cp_in.start()
        cp_in.wait()
        ob[...] = jnp.dot(xb[...], gb[...], preferred_element_type=jnp.float32)
        cp_out = pltpu.make_async_copy(
            ob, o_hbm.at[pl.ds(i * BLK, BLK), :], semo
        )
        cp_out.start()
        cp_out.wait()


def kernel(X):
    spec, var, flag = pl.pallas_call(
        _pass1,
        grid=(NBLK,),
        in_specs=[pl.BlockSpec((BLK, N_COLS), lambda i: (i, 0))],
        out_specs=[
            pl.BlockSpec((BLK, N_COLS), lambda i: (i, 0)),
            pl.BlockSpec((1, N_COLS), lambda i: (0, 0)),
            pl.BlockSpec((1, 1), lambda i: (0, 0)),
        ],
        out_shape=[
            jax.ShapeDtypeStruct((N_ROWS, N_COLS), jnp.float32),
            jax.ShapeDtypeStruct((1, N_COLS), jnp.float32),
            jax.ShapeDtypeStruct((1, 1), jnp.int32),
        ],
        scratch_shapes=[
            pltpu.VMEM((8, N_COLS), jnp.float32),
            pltpu.VMEM((8, N_COLS), jnp.float32),
        ],
        compiler_params=pltpu.CompilerParams(
            dimension_semantics=("arbitrary",),
        ),
    )(X)

    out = pl.pallas_call(
        _pass2,
        grid=(NBLK,),
        in_specs=[
            pl.BlockSpec(memory_space=pl.ANY),
            pl.BlockSpec(memory_space=pl.ANY),
            pl.BlockSpec((1, N_COLS), lambda i: (0, 0)),
            pl.BlockSpec((1, 1), lambda i: (0, 0), memory_space=pltpu.SMEM),
        ],
        out_specs=pl.BlockSpec(memory_space=pl.ANY),
        out_shape=jax.ShapeDtypeStruct((N_ROWS, N_COLS), jnp.float32),
        input_output_aliases={1: 0},
        scratch_shapes=[
            pltpu.VMEM((BLK, N_COLS), jnp.float32),
            pltpu.VMEM((BLK, N_COLS), jnp.float32),
            pltpu.VMEM((N_COLS, N_COLS), jnp.float32),
            pltpu.SemaphoreType.DMA,
            pltpu.SemaphoreType.DMA,
        ],
        compiler_params=pltpu.CompilerParams(
            dimension_semantics=("arbitrary",),
        ),
    )(X, spec, var, flag)
    return out


# speedup vs baseline: 5.6167x; 1.1164x over previous
import jax
import jax.numpy as jnp
from jax.experimental import pallas as pl
from jax.experimental.pallas import tpu as pltpu

N_ROWS = 16384
N_COLS = 2048
BLK = 512
NBLK = N_ROWS // BLK
FBLK = 256
NFIX = N_ROWS // FBLK
NF = float(N_ROWS)


def _pass1(x_ref, o_ref, var_ref, flag_ref, acc_sum, acc_sq):
    i = pl.program_id(0)

    @pl.when(i == 0)
    def _init():
        acc_sum[...] = jnp.zeros_like(acc_sum)
        acc_sq[...] = jnp.zeros_like(acc_sq)

    x = x_ref[...]
    o_ref[...] = x
    xr = x.reshape(BLK // 8, 8, N_COLS)
    acc_sum[...] += jnp.sum(xr, axis=0)
    acc_sq[...] += jnp.sum(xr * xr, axis=0)

    @pl.when(i == NBLK - 1)
    def _finish():
        s = jnp.sum(acc_sum[...], axis=0, keepdims=True)
        q = jnp.sum(acc_sq[...], axis=0, keepdims=True)
        var = (q - s * s / NF) / (NF - 1.0)
        var_ref[...] = var
        m = (var > 0.0).astype(jnp.float32)
        cnt = jnp.sum(m, axis=1, keepdims=True)
        flag_ref[...] = (cnt == float(N_COLS)).astype(jnp.int32)


def _pass2(x_hbm, spec_hbm, var_ref, flag_ref, o_hbm, xb, ob, gb, semx, semo):
    slow = flag_ref[0, 0] == 0

    @pl.when(slow)
    def _build_g():
        var = var_ref[...]
        m = (var > 0.0).astype(jnp.float32)
        cs = m
        sh = 1
        while sh < N_COLS:
            cs = cs + jnp.concatenate(
                [jnp.zeros((1, sh), jnp.float32), cs[:, :-sh]], axis=1
            )
            sh *= 2
        pos = (cs - 1.0).astype(jnp.int32)
        cnt = cs[:, N_COLS - 1:].astype(jnp.int32)
        io0 = jax.lax.broadcasted_iota(jnp.int32, (N_COLS, N_COLS), 0)
        io1 = jax.lax.broadcasted_iota(jnp.int32, (N_COLS, N_COLS), 1)
        keep = (io0 == pos) & (m > 0.0)
        pad = (io1 == 0) & (io0 >= cnt)
        gt = jnp.where(keep | pad, 1.0, 0.0)
        gb[...] = gt.T

        def _gather_block(i, carry):
            cp_in = pltpu.make_async_copy(
                x_hbm.at[pl.ds(i * FBLK, FBLK), :], xb, semx
            )
            cp_in.start()
            cp_in.wait()
            ob[...] = jnp.dot(
                xb[...], gb[...], preferred_element_type=jnp.float32
            )
            cp_out = pltpu.make_async_copy(
                ob, o_hbm.at[pl.ds(i * FBLK, FBLK), :], semo
            )
            cp_out.start()
            cp_out.wait()
            return carry

        jax.lax.fori_loop(0, NFIX, _gather_block, 0)


def kernel(X):
    spec, var, flag = pl.pallas_call(
        _pass1,
        grid=(NBLK,),
        in_specs=[pl.BlockSpec((BLK, N_COLS), lambda i: (i, 0))],
        out_specs=[
            pl.BlockSpec((BLK, N_COLS), lambda i: (i, 0)),
            pl.BlockSpec((1, N_COLS), lambda i: (0, 0)),
            pl.BlockSpec((1, 1), lambda i: (0, 0)),
        ],
        out_shape=[
            jax.ShapeDtypeStruct((N_ROWS, N_COLS), jnp.float32),
            jax.ShapeDtypeStruct((1, N_COLS), jnp.float32),
            jax.ShapeDtypeStruct((1, 1), jnp.int32),
        ],
        scratch_shapes=[
            pltpu.VMEM((8, N_COLS), jnp.float32),
            pltpu.VMEM((8, N_COLS), jnp.float32),
        ],
        compiler_params=pltpu.CompilerParams(
            dimension_semantics=("arbitrary",),
        ),
    )(X)

    out = pl.pallas_call(
        _pass2,
        in_specs=[
            pl.BlockSpec(memory_space=pl.ANY),
            pl.BlockSpec(memory_space=pl.ANY),
            pl.BlockSpec((1, N_COLS), lambda: (0, 0)),
            pl.BlockSpec((1, 1), lambda: (0, 0), memory_space=pltpu.SMEM),
        ],
        out_specs=pl.BlockSpec(memory_space=pl.ANY),
        out_shape=jax.ShapeDtypeStruct((N_ROWS, N_COLS), jnp.float32),
        input_output_aliases={1: 0},
        scratch_shapes=[
            pltpu.VMEM((FBLK, N_COLS), jnp.float32),
            pltpu.VMEM((FBLK, N_COLS), jnp.float32),
            pltpu.VMEM((N_COLS, N_COLS), jnp.float32),
            pltpu.SemaphoreType.DMA,
            pltpu.SemaphoreType.DMA,
        ],
    )(X, spec, var, flag)
    return out


# speedup vs baseline: 5.7422x; 1.0224x over previous
import jax
import jax.numpy as jnp
from jax.experimental import pallas as pl
from jax.experimental.pallas import tpu as pltpu

N_ROWS = 16384
N_COLS = 2048
BLK = 1024
NBLK = N_ROWS // BLK
FBLK = 256
NFIX = N_ROWS // FBLK
NF = float(N_ROWS)


def _pass1(x_ref, o_ref, var_ref, flag_ref, acc_sum, acc_sq):
    i = pl.program_id(0)

    @pl.when(i == 0)
    def _init():
        acc_sum[...] = jnp.zeros_like(acc_sum)
        acc_sq[...] = jnp.zeros_like(acc_sq)

    x = x_ref[...]
    o_ref[...] = x
    xr = x.reshape(BLK // 8, 8, N_COLS)
    acc_sum[...] += jnp.sum(xr, axis=0)
    acc_sq[...] += jnp.sum(xr * xr, axis=0)

    @pl.when(i == NBLK - 1)
    def _finish():
        s = jnp.sum(acc_sum[...], axis=0, keepdims=True)
        q = jnp.sum(acc_sq[...], axis=0, keepdims=True)
        var = (q - s * s / NF) / (NF - 1.0)
        var_ref[...] = var
        m = (var > 0.0).astype(jnp.float32)
        cnt = jnp.sum(m, axis=1, keepdims=True)
        flag_ref[...] = (cnt == float(N_COLS)).astype(jnp.int32)


def _pass2(x_hbm, spec_hbm, var_ref, flag_ref, o_hbm, xb, ob, gb, semx, semo):
    slow = flag_ref[0, 0] == 0

    @pl.when(slow)
    def _build_g():
        var = var_ref[...]
        m = (var > 0.0).astype(jnp.float32)
        cs = m
        sh = 1
        while sh < N_COLS:
            cs = cs + jnp.concatenate(
                [jnp.zeros((1, sh), jnp.float32), cs[:, :-sh]], axis=1
            )
            sh *= 2
        pos = (cs - 1.0).astype(jnp.int32)
        cnt = cs[:, N_COLS - 1:].astype(jnp.int32)
        io0 = jax.lax.broadcasted_iota(jnp.int32, (N_COLS, N_COLS), 0)
        io1 = jax.lax.broadcasted_iota(jnp.int32, (N_COLS, N_COLS), 1)
        keep = (io0 == pos) & (m > 0.0)
        pad = (io1 == 0) & (io0 >= cnt)
        gt = jnp.where(keep | pad, 1.0, 0.0)
        gb[...] = gt.T

        def _gather_block(i, carry):
            cp_in = pltpu.make_async_copy(
                x_hbm.at[pl.ds(i * FBLK, FBLK), :], xb, semx
            )
            cp_in.start()
            cp_in.wait()
            ob[...] = jnp.dot(
                xb[...], gb[...], preferred_element_type=jnp.float32
            )
            cp_out = pltpu.make_async_copy(
                ob, o_hbm.at[pl.ds(i * FBLK, FBLK), :], semo
            )
            cp_out.start()
            cp_out.wait()
            return carry

        jax.lax.fori_loop(0, NFIX, _gather_block, 0)


def kernel(X):
    spec, var, flag = pl.pallas_call(
        _pass1,
        grid=(NBLK,),
        in_specs=[pl.BlockSpec((BLK, N_COLS), lambda i: (i, 0))],
        out_specs=[
            pl.BlockSpec((BLK, N_COLS), lambda i: (i, 0)),
            pl.BlockSpec((1, N_COLS), lambda i: (0, 0)),
            pl.BlockSpec((1, 1), lambda i: (0, 0)),
        ],
        out_shape=[
            jax.ShapeDtypeStruct((N_ROWS, N_COLS), jnp.float32),
            jax.ShapeDtypeStruct((1, N_COLS), jnp.float32),
            jax.ShapeDtypeStruct((1, 1), jnp.int32),
        ],
        scratch_shapes=[
            pltpu.VMEM((8, N_COLS), jnp.float32),
            pltpu.VMEM((8, N_COLS), jnp.float32),
        ],
        compiler_params=pltpu.CompilerParams(
            dimension_semantics=("arbitrary",),
        ),
    )(X)

    out = pl.pallas_call(
        _pass2,
        in_specs=[
            pl.BlockSpec(memory_space=pl.ANY),
            pl.BlockSpec(memory_space=pl.ANY),
            pl.BlockSpec((1, N_COLS), lambda: (0, 0)),
            pl.BlockSpec((1, 1), lambda: (0, 0), memory_space=pltpu.SMEM),
        ],
        out_specs=pl.BlockSpec(memory_space=pl.ANY),
        out_shape=jax.ShapeDtypeStruct((N_ROWS, N_COLS), jnp.float32),
        input_output_aliases={1: 0},
        scratch_shapes=[
            pltpu.VMEM((FBLK, N_COLS), jnp.float32),
            pltpu.VMEM((FBLK, N_COLS), jnp.float32),
            pltpu.VMEM((N_COLS, N_COLS), jnp.float32),
            pltpu.SemaphoreType.DMA,
            pltpu.SemaphoreType.DMA,
        ],
    )(X, spec, var, flag)
    return out
